# baseline (speedup 1.0000x reference)
import functools

import jax
import jax.numpy as jnp
from jax import lax
from jax.experimental import pallas as pl
from jax.experimental.pallas import tpu as pltpu
from jax.experimental.pallas import tpu_sc as plsc

N = 10000
E = 320000
IN_DIM = 128
HD = 128

NC = 2
NS = 16
NP = 10240
CHUNK = 128
EROWS = 2560
RPW = EROWS // (NC * NS)
GRP = 16
STRIPE = NP // NS
BLK = 1024
GRID = NP // BLK


def _pre_body(x_ref, ws_ref, wd_ref, a1_ref, a2_ref,
              h1_ref, h2_ref, l1s_ref, l1d_ref, l2s_ref, l2d_ref):
    x = x_ref[...]
    h1 = jnp.dot(x, ws_ref[...], preferred_element_type=jnp.float32)
    h2 = jnp.dot(x, wd_ref[...], preferred_element_type=jnp.float32)
    h1_ref[...] = h1
    h2_ref[...] = h2
    a1 = jnp.dot(h1, a1_ref[...], preferred_element_type=jnp.float32)
    a2 = jnp.dot(h2, a2_ref[...], preferred_element_type=jnp.float32)
    l1s_ref[...] = a1[:, :16]
    l1d_ref[...] = a1[:, 16:]
    l2s_ref[...] = a2[:, :16]
    l2d_ref[...] = a2[:, 16:]


def _pre_call(xp, w_src, w_dst, a1, a2):
    f32 = jnp.float32
    return pl.pallas_call(
        _pre_body,
        grid=(GRID,),
        in_specs=[
            pl.BlockSpec((BLK, IN_DIM), lambda i: (i, 0)),
            pl.BlockSpec((IN_DIM, HD), lambda i: (0, 0)),
            pl.BlockSpec((IN_DIM, HD), lambda i: (0, 0)),
            pl.BlockSpec((IN_DIM, 32), lambda i: (0, 0)),
            pl.BlockSpec((IN_DIM, 32), lambda i: (0, 0)),
        ],
        out_specs=[
            pl.BlockSpec((BLK, HD), lambda i: (i, 0)),
            pl.BlockSpec((BLK, HD), lambda i: (i, 0)),
            pl.BlockSpec((BLK, 16), lambda i: (i, 0)),
            pl.BlockSpec((BLK, 16), lambda i: (i, 0)),
            pl.BlockSpec((BLK, 16), lambda i: (i, 0)),
            pl.BlockSpec((BLK, 16), lambda i: (i, 0)),
        ],
        out_shape=[
            jax.ShapeDtypeStruct((NP, HD), f32),
            jax.ShapeDtypeStruct((NP, HD), f32),
            jax.ShapeDtypeStruct((NP, 16), f32),
            jax.ShapeDtypeStruct((NP, 16), f32),
            jax.ShapeDtypeStruct((NP, 16), f32),
            jax.ShapeDtypeStruct((NP, 16), f32),
        ],
    )(xp, w_src, w_dst, a1, a2)


def _sc_body(h_hbm, ls_hbm, ld_hbm, src_hbm, dst_hbm,
             out_hbm, den_hbm,
             out_acc, den_acc, src_v, dst_v, rows, asb, adb, wb,
             sem1, sem2):
    c = lax.axis_index("c")
    s = lax.axis_index("s")
    wid = c * NS + s
    zero16 = jnp.zeros((16,), jnp.float32)

    def zbody(e, carry):
        wb[e, :] = zero16
        for g in range(8):
            rows[e, pl.ds(16 * g, 16)] = zero16
        return carry

    lax.fori_loop(0, CHUNK, zbody, 0)
    base = s * STRIPE
    for k in range(STRIPE // CHUNK):
        pltpu.sync_copy(rows, out_acc.at[pl.ds(base + k * CHUNK, CHUNK)])
        pltpu.sync_copy(wb, den_acc.at[pl.ds(base + k * CHUNK, CHUNK)])

    plsc.subcore_barrier()

    def group_body(gg, carry):
        gbase = wid * RPW + gg * GRP
        pltpu.sync_copy(src_hbm.at[pl.ds(gbase, GRP)], src_v)
        pltpu.sync_copy(dst_hbm.at[pl.ds(gbase, GRP)], dst_v)

        def chunk_body(j, carry1):
            isrc = src_v.at[j]
            idst = dst_v.at[j]
            cp_rows = pltpu.async_copy(h_hbm.at[isrc], rows, sem1)
            cp_s = pltpu.async_copy(ls_hbm.at[isrc], asb, sem2)
            cp_d = pltpu.async_copy(ld_hbm.at[idst], adb, sem2)
            cp_s.wait()
            cp_d.wait()

            def wbody(e, carry2):
                av = asb[e, :] + adb[e, :]
                wb[e, :] = jnp.exp(jnp.maximum(av, 0.2 * av))
                return carry2

            lax.fori_loop(0, CHUNK, wbody, 0, unroll=2)
            cp_rows.wait()

            def ebody(e, carry2):
                w = wb[e, :]
                for g in range(8):
                    sl = pl.ds(16 * g, 16)
                    rows[e, sl] = rows[e, sl] * w[g]
                return carry2

            lax.fori_loop(0, CHUNK, ebody, 0, unroll=2)
            pltpu.sync_copy(rows, out_acc.at[idst], add=True)
            pltpu.sync_copy(wb, den_acc.at[idst], add=True)
            return carry1

        lax.fori_loop(0, GRP, chunk_body, 0)
        return carry

    lax.fori_loop(0, RPW // GRP, group_body, 0)
    plsc.subcore_barrier()

    pltpu.sync_copy(out_acc.at[pl.ds(base, STRIPE)],
                    out_hbm.at[c, pl.ds(base, STRIPE)])
    pltpu.sync_copy(den_acc.at[pl.ds(base, STRIPE)],
                    den_hbm.at[c, pl.ds(base, STRIPE)])


@functools.cache
def _sc_layer_call():
    f32 = jnp.float32
    mesh = plsc.VectorSubcoreMesh(
        core_axis_name="c", subcore_axis_name="s",
        num_cores=NC, num_subcores=NS)
    return pl.kernel(
        _sc_body,
        out_type=(
            jax.ShapeDtypeStruct((NC, NP, HD), f32),
            jax.ShapeDtypeStruct((NC, NP, 16), f32),
        ),
        mesh=mesh,
        compiler_params=pltpu.CompilerParams(use_tc_tiling_on_sc=False),
        scratch_types=[
            pltpu.VMEM_SHARED((NP, HD), f32),
            pltpu.VMEM_SHARED((NP, 16), f32),
            pltpu.VMEM((GRP, CHUNK), jnp.int32),
            pltpu.VMEM((GRP, CHUNK), jnp.int32),
            pltpu.VMEM((CHUNK, HD), f32),
            pltpu.VMEM((CHUNK, 16), f32),
            pltpu.VMEM((CHUNK, 16), f32),
            pltpu.VMEM((CHUNK, 16), f32),
            pltpu.SemaphoreType.DMA,
            pltpu.SemaphoreType.DMA,
        ],
    )


def _post_body(o1_ref, d1_ref, o2_ref, d2_ref, r_ref, y_ref):
    r = r_ref[...]
    o1 = o1_ref[0] + o1_ref[1]
    o2 = o2_ref[0] + o2_ref[1]
    d1 = jnp.dot(d1_ref[0] + d1_ref[1], r,
                 preferred_element_type=jnp.float32) + 1e-9
    d2 = jnp.dot(d2_ref[0] + d2_ref[1], r,
                 preferred_element_type=jnp.float32) + 1e-9
    t = o1 / d1 + o2 / d2
    y_ref[...] = jnp.where(t > 0.0, t, jnp.exp(jnp.minimum(t, 0.0)) - 1.0)


def _post_call(o1, d1, o2, d2, rmat):
    return pl.pallas_call(
        _post_body,
        grid=(GRID,),
        in_specs=[
            pl.BlockSpec((NC, BLK, HD), lambda i: (0, i, 0)),
            pl.BlockSpec((NC, BLK, 16), lambda i: (0, i, 0)),
            pl.BlockSpec((NC, BLK, HD), lambda i: (0, i, 0)),
            pl.BlockSpec((NC, BLK, 16), lambda i: (0, i, 0)),
            pl.BlockSpec((16, HD), lambda i: (0, 0)),
        ],
        out_specs=pl.BlockSpec((BLK, HD), lambda i: (i, 0)),
        out_shape=jax.ShapeDtypeStruct((NP, HD), jnp.float32),
    )(o1, d1, o2, d2, rmat)


def _alpha_mat(a_s, a_d):
    am = jnp.zeros((IN_DIM, 32), jnp.float32)
    rr = jnp.arange(IN_DIM)
    hh = rr // 16
    am = am.at[rr, hh].set(a_s.astype(jnp.float32).reshape(-1))
    am = am.at[rr, 16 + hh].set(a_d.astype(jnp.float32).reshape(-1))
    return am


def _rep_mat():
    jj = jnp.arange(HD)
    return jnp.zeros((16, HD), jnp.float32).at[jj // 16, jj].set(1.0)


def _pad_edges(v):
    npad = EROWS * CHUNK - E
    pad = N + jnp.arange(npad, dtype=jnp.int32) % (NP - N)
    return jnp.concatenate([v.astype(jnp.int32), pad]).reshape(EROWS, CHUNK)


def kernel(x, edge_index, W_src, a_src_s, a_src_d, W_dst, a_dst_s, a_dst_d):
    src = _pad_edges(edge_index[0])
    dst = _pad_edges(edge_index[1])
    xp = jnp.pad(x, ((0, NP - N), (0, 0)))
    a1 = _alpha_mat(a_src_s, a_src_d)
    a2 = _alpha_mat(a_dst_s, a_dst_d)
    h1, h2, l1s, l1d, l2s, l2d = _pre_call(xp, W_src, W_dst, a1, a2)
    sc = _sc_layer_call()
    out1, den1 = sc(h1, l1s, l1d, src, dst)
    out2, den2 = sc(h2, l2s, l2d, dst, src)
    y = _post_call(out1, den1, out2, den2, _rep_mat())
    return y[:N]


# speedup vs baseline: 1.2561x; 1.2561x over previous
import functools

import jax
import jax.numpy as jnp
from jax import lax
from jax.experimental import pallas as pl
from jax.experimental.pallas import tpu as pltpu
from jax.experimental.pallas import tpu_sc as plsc

N = 10000
E = 320000
IN_DIM = 128
HD = 128

NC = 2
NS = 16
NP = 10240
CHUNK = 128
EROWS = 2560
RPW = EROWS // (NC * NS)
GRP = 16
STRIPE = NP // NS
BLK = 1024
GRID = NP // BLK


def _pre_body(x_ref, ws_ref, wd_ref, a1_ref, a2_ref,
              h1_ref, h2_ref, l1s_ref, l1d_ref, l2s_ref, l2d_ref):
    x = x_ref[...]
    h1 = jnp.dot(x, ws_ref[...], preferred_element_type=jnp.float32)
    h2 = jnp.dot(x, wd_ref[...], preferred_element_type=jnp.float32)
    h1_ref[...] = h1
    h2_ref[...] = h2
    a1 = jnp.dot(h1, a1_ref[...], preferred_element_type=jnp.float32)
    a2 = jnp.dot(h2, a2_ref[...], preferred_element_type=jnp.float32)
    l1s_ref[...] = a1[:, :16]
    l1d_ref[...] = a1[:, 16:]
    l2s_ref[...] = a2[:, :16]
    l2d_ref[...] = a2[:, 16:]


def _pre_call(xp, w_src, w_dst, a1, a2):
    f32 = jnp.float32
    return pl.pallas_call(
        _pre_body,
        grid=(GRID,),
        in_specs=[
            pl.BlockSpec((BLK, IN_DIM), lambda i: (i, 0)),
            pl.BlockSpec((IN_DIM, HD), lambda i: (0, 0)),
            pl.BlockSpec((IN_DIM, HD), lambda i: (0, 0)),
            pl.BlockSpec((IN_DIM, 32), lambda i: (0, 0)),
            pl.BlockSpec((IN_DIM, 32), lambda i: (0, 0)),
        ],
        out_specs=[
            pl.BlockSpec((BLK, HD), lambda i: (i, 0)),
            pl.BlockSpec((BLK, HD), lambda i: (i, 0)),
            pl.BlockSpec((BLK, 16), lambda i: (i, 0)),
            pl.BlockSpec((BLK, 16), lambda i: (i, 0)),
            pl.BlockSpec((BLK, 16), lambda i: (i, 0)),
            pl.BlockSpec((BLK, 16), lambda i: (i, 0)),
        ],
        out_shape=[
            jax.ShapeDtypeStruct((NP, HD), f32),
            jax.ShapeDtypeStruct((NP, HD), f32),
            jax.ShapeDtypeStruct((NP, 16), f32),
            jax.ShapeDtypeStruct((NP, 16), f32),
            jax.ShapeDtypeStruct((NP, 16), f32),
            jax.ShapeDtypeStruct((NP, 16), f32),
        ],
    )(xp, w_src, w_dst, a1, a2)


def _sc_body(h_hbm, ls_hbm, ld_hbm, src_hbm, dst_hbm,
             out_hbm, den_hbm,
             out_acc, den_acc, src_v, dst_v, rows, asb, adb, wb,
             sem1, sem2):
    c = lax.axis_index("c")
    s = lax.axis_index("s")
    wid = c * NS + s
    zero16 = jnp.zeros((16,), jnp.float32)

    def zbody(e, carry):
        wb[e, :] = zero16
        for g in range(8):
            rows[e, pl.ds(16 * g, 16)] = zero16
        return carry

    lax.fori_loop(0, CHUNK, zbody, 0)
    base = s * STRIPE
    for k in range(STRIPE // CHUNK):
        pltpu.sync_copy(rows, out_acc.at[pl.ds(base + k * CHUNK, CHUNK)])
        pltpu.sync_copy(wb, den_acc.at[pl.ds(base + k * CHUNK, CHUNK)])

    plsc.subcore_barrier()

    def group_body(gg, carry):
        gbase = wid * RPW + gg * GRP
        pltpu.sync_copy(src_hbm.at[pl.ds(gbase, GRP)], src_v)
        pltpu.sync_copy(dst_hbm.at[pl.ds(gbase, GRP)], dst_v)

        def chunk_body(j, carry1):
            isrc = src_v.at[j]
            idst = dst_v.at[j]
            cp_rows = pltpu.async_copy(h_hbm.at[isrc], rows, sem1)
            cp_s = pltpu.async_copy(ls_hbm.at[isrc], asb, sem2)
            cp_d = pltpu.async_copy(ld_hbm.at[idst], adb, sem2)
            cp_s.wait()
            cp_d.wait()

            def wbody(e, carry2):
                av = asb[e, :] + adb[e, :]
                wb[e, :] = jnp.exp(jnp.maximum(av, 0.2 * av))
                return carry2

            lax.fori_loop(0, CHUNK, wbody, 0)
            cp_rows.wait()

            def ebody(e, carry2):
                w = wb[e, :]
                for g in range(8):
                    sl = pl.ds(16 * g, 16)
                    rows[e, sl] = rows[e, sl] * w[g]
                return carry2

            lax.fori_loop(0, CHUNK, ebody, 0)
            pltpu.sync_copy(rows, out_acc.at[idst], add=True)
            pltpu.sync_copy(wb, den_acc.at[idst], add=True)
            return carry1

        lax.fori_loop(0, GRP, chunk_body, 0)
        return carry

    lax.fori_loop(0, RPW // GRP, group_body, 0)
    plsc.subcore_barrier()

    pltpu.sync_copy(out_acc.at[pl.ds(base, STRIPE)],
                    out_hbm.at[c, pl.ds(base, STRIPE)])
    pltpu.sync_copy(den_acc.at[pl.ds(base, STRIPE)],
                    den_hbm.at[c, pl.ds(base, STRIPE)])


@functools.cache
def _sc_layer_call():
    f32 = jnp.float32
    mesh = plsc.VectorSubcoreMesh(
        core_axis_name="c", subcore_axis_name="s",
        num_cores=NC, num_subcores=NS)
    return pl.kernel(
        _sc_body,
        out_type=(
            jax.ShapeDtypeStruct((NC, NP, HD), f32),
            jax.ShapeDtypeStruct((NC, NP, 16), f32),
        ),
        mesh=mesh,
        compiler_params=pltpu.CompilerParams(use_tc_tiling_on_sc=False),
        scratch_types=[
            pltpu.VMEM_SHARED((NP, HD), f32),
            pltpu.VMEM_SHARED((NP, 16), f32),
            pltpu.VMEM((GRP, CHUNK), jnp.int32),
            pltpu.VMEM((GRP, CHUNK), jnp.int32),
            pltpu.VMEM((CHUNK, HD), f32),
            pltpu.VMEM((CHUNK, 16), f32),
            pltpu.VMEM((CHUNK, 16), f32),
            pltpu.VMEM((CHUNK, 16), f32),
            pltpu.SemaphoreType.DMA,
            pltpu.SemaphoreType.DMA,
        ],
    )


def _post_body(o1_ref, d1_ref, o2_ref, d2_ref, r_ref, y_ref):
    r = r_ref[...]
    o1 = o1_ref[0] + o1_ref[1]
    o2 = o2_ref[0] + o2_ref[1]
    d1 = jnp.dot(d1_ref[0] + d1_ref[1], r,
                 preferred_element_type=jnp.float32) + 1e-9
    d2 = jnp.dot(d2_ref[0] + d2_ref[1], r,
                 preferred_element_type=jnp.float32) + 1e-9
    t = o1 / d1 + o2 / d2
    y_ref[...] = jnp.where(t > 0.0, t, jnp.exp(jnp.minimum(t, 0.0)) - 1.0)


def _post_call(o1, d1, o2, d2, rmat):
    return pl.pallas_call(
        _post_body,
        grid=(GRID,),
        in_specs=[
            pl.BlockSpec((NC, BLK, HD), lambda i: (0, i, 0)),
            pl.BlockSpec((NC, BLK, 16), lambda i: (0, i, 0)),
            pl.BlockSpec((NC, BLK, HD), lambda i: (0, i, 0)),
            pl.BlockSpec((NC, BLK, 16), lambda i: (0, i, 0)),
            pl.BlockSpec((16, HD), lambda i: (0, 0)),
        ],
        out_specs=pl.BlockSpec((BLK, HD), lambda i: (i, 0)),
        out_shape=jax.ShapeDtypeStruct((NP, HD), jnp.float32),
    )(o1, d1, o2, d2, rmat)


def _alpha_mat(a_s, a_d):
    am = jnp.zeros((IN_DIM, 32), jnp.float32)
    rr = jnp.arange(IN_DIM)
    hh = rr // 16
    am = am.at[rr, hh].set(a_s.astype(jnp.float32).reshape(-1))
    am = am.at[rr, 16 + hh].set(a_d.astype(jnp.float32).reshape(-1))
    return am


def _rep_mat():
    jj = jnp.arange(HD)
    return jnp.zeros((16, HD), jnp.float32).at[jj // 16, jj].set(1.0)


def _pad_edges(v):
    npad = EROWS * CHUNK - E
    pad = N + jnp.arange(npad, dtype=jnp.int32) % (NP - N)
    return jnp.concatenate([v.astype(jnp.int32), pad]).reshape(EROWS, CHUNK)


def kernel(x, edge_index, W_src, a_src_s, a_src_d, W_dst, a_dst_s, a_dst_d):
    src = _pad_edges(edge_index[0])
    dst = _pad_edges(edge_index[1])
    xp = jnp.pad(x, ((0, NP - N), (0, 0)))
    a1 = _alpha_mat(a_src_s, a_src_d)
    a2 = _alpha_mat(a_dst_s, a_dst_d)
    h1, h2, l1s, l1d, l2s, l2d = _pre_call(xp, W_src, W_dst, a1, a2)
    sc = _sc_layer_call()
    out1, den1 = sc(h1, l1s, l1d, src, dst)
    out2, den2 = sc(h2, l2s, l2d, dst, src)
    y = _post_call(out1, den1, out2, den2, _rep_mat())
    return y[:N]
